# baseline (speedup 1.0000x reference)
import functools

import jax
import jax.numpy as jnp
from jax import lax
from jax.experimental import pallas as pl
from jax.experimental.pallas import tpu as pltpu
from jax.experimental.pallas import tpu_sc as plsc

_LANES = 16
_NWORKERS = 16


def _mlp_body(emb_ref, w1_ref, b1_ref, w2t_ref, b2_ref, out_ref):
    h = jnp.dot(emb_ref[...], w1_ref[...], preferred_element_type=jnp.float32)
    h = h + b1_ref[...]
    h = h * jax.nn.sigmoid(h)
    tab = lax.dot_general(w2t_ref[...], h, (((1,), (1,)), ((), ())),
                          preferred_element_type=jnp.float32)
    out_ref[...] = tab + b2_ref[0, 0]


def _vocab_mlp(emb_table, W1, b1, W2, b2):
    vocab = emb_table.shape[0]
    tab2 = pl.pallas_call(
        _mlp_body,
        out_shape=jax.ShapeDtypeStruct((1, vocab), jnp.float32),
    )(emb_table, W1, b1.reshape(1, -1), W2.reshape(1, -1), b2.reshape(1, 1))
    return tab2.reshape(vocab)


def _gather_loop(tab_v, idx_v, val_v, count, unroll):
    def body(i, carry):
        s = i * (_LANES * unroll)
        for u in range(unroll):
            o = s + u * _LANES
            idx = idx_v[pl.ds(o, _LANES)]
            val_v[pl.ds(o, _LANES)] = plsc.load_gather(tab_v, [idx])
        return carry

    lax.fori_loop(0, count // unroll, body, 0)


def _make_sc_gather(n, vocab):
    chunk = -(-n // _NWORKERS)
    chunk = -(-chunk // (8 * _LANES)) * (8 * _LANES)
    assert chunk <= n and chunk % (8 * _LANES) == 0
    assert (n - chunk) % _LANES == 0

    mesh = plsc.VectorSubcoreMesh(core_axis_name="c", subcore_axis_name="s", num_cores=1)

    @functools.partial(
        pl.kernel,
        out_type=jax.ShapeDtypeStruct((n,), jnp.float32),
        mesh=mesh,
        scratch_types=[
            pltpu.VMEM((chunk,), jnp.int32),
            pltpu.VMEM((chunk,), jnp.float32),
            pltpu.VMEM((vocab,), jnp.float32),
            pltpu.SemaphoreType.DMA,
        ],
        compiler_params=pltpu.CompilerParams(needs_layout_passes=False),
    )
    def sc_gather(z_hbm, tab_hbm, out_hbm, idx_v, val_v, tab_v, sem):
        wid = lax.axis_index("s")
        base = jnp.minimum(wid * chunk, n - chunk)
        cp = pltpu.async_copy(z_hbm.at[pl.ds(base, chunk)], idx_v, sem)
        pltpu.sync_copy(tab_hbm, tab_v)
        cp.wait()
        _gather_loop(tab_v, idx_v, val_v, chunk // _LANES, 8)
        pltpu.sync_copy(val_v, out_hbm.at[pl.ds(base, chunk)])

    return sc_gather


def kernel(z, batch, pos, emb_table, W1, b1, W2, b2):
    n = z.shape[0]
    vocab = emb_table.shape[0]
    tab = _vocab_mlp(emb_table, W1, b1, W2, b2)
    outp = _make_sc_gather(n, vocab)(z.astype(jnp.int32), tab)
    return outp.reshape(n, 1)


# speedup vs baseline: 1.0011x; 1.0011x over previous
import functools

import jax
import jax.numpy as jnp
from jax import lax
from jax.experimental import pallas as pl
from jax.experimental.pallas import tpu as pltpu
from jax.experimental.pallas import tpu_sc as plsc

_LANES = 16
_NWORKERS = 16


def _mlp_body(emb_ref, w1_ref, b1_ref, w2t_ref, b2_ref, out_ref):
    h = jnp.dot(emb_ref[...], w1_ref[...], preferred_element_type=jnp.float32)
    h = h + b1_ref[...]
    h = h * jax.nn.sigmoid(h)
    tab = lax.dot_general(w2t_ref[...], h, (((1,), (1,)), ((), ())),
                          preferred_element_type=jnp.float32)
    out_ref[...] = tab + b2_ref[0, 0]


def _vocab_mlp(emb_table, W1, b1, W2, b2):
    vocab = emb_table.shape[0]
    tab2 = pl.pallas_call(
        _mlp_body,
        out_shape=jax.ShapeDtypeStruct((1, vocab), jnp.float32),
    )(emb_table, W1, b1.reshape(1, -1), W2.reshape(1, -1), b2.reshape(1, 1))
    return tab2.reshape(vocab)


def _gather_loop(tab_v, idx_v, val_v, count, unroll):
    def body(i, carry):
        s = i * (_LANES * unroll)
        for u in range(unroll):
            o = s + u * _LANES
            idx = idx_v[pl.ds(o, _LANES)]
            val_v[pl.ds(o, _LANES)] = plsc.load_gather(tab_v, [idx])
        return carry

    lax.fori_loop(0, count // unroll, body, 0)


def _make_sc_gather(n, vocab):
    chunk = -(-n // _NWORKERS)
    chunk = -(-chunk // (4 * _LANES)) * (4 * _LANES)
    assert chunk <= n and chunk % (4 * _LANES) == 0
    assert (n - chunk) % _LANES == 0

    mesh = plsc.VectorSubcoreMesh(core_axis_name="c", subcore_axis_name="s", num_cores=1)

    @functools.partial(
        pl.kernel,
        out_type=jax.ShapeDtypeStruct((n,), jnp.float32),
        mesh=mesh,
        scratch_types=[
            pltpu.VMEM((chunk,), jnp.int32),
            pltpu.VMEM((chunk,), jnp.float32),
            pltpu.VMEM((vocab,), jnp.float32),
            pltpu.SemaphoreType.DMA,
        ],
        compiler_params=pltpu.CompilerParams(needs_layout_passes=False),
    )
    def sc_gather(z_hbm, tab_hbm, out_hbm, idx_v, val_v, tab_v, sem):
        wid = lax.axis_index("s")
        base = jnp.minimum(wid * chunk, n - chunk)
        cp = pltpu.async_copy(z_hbm.at[pl.ds(base, chunk)], idx_v, sem)
        pltpu.sync_copy(tab_hbm, tab_v)
        cp.wait()
        _gather_loop(tab_v, idx_v, val_v, chunk // _LANES, 4)
        pltpu.sync_copy(val_v, out_hbm.at[pl.ds(base, chunk)])

    return sc_gather


def kernel(z, batch, pos, emb_table, W1, b1, W2, b2):
    n = z.shape[0]
    vocab = emb_table.shape[0]
    tab = _vocab_mlp(emb_table, W1, b1, W2, b2)
    outp = _make_sc_gather(n, vocab)(z.astype(jnp.int32), tab)
    return outp.reshape(n, 1)
